# baseline (speedup 1.0000x reference)
import functools

import jax
import jax.numpy as jnp
from jax import lax
from jax.experimental import pallas as pl
from jax.experimental.pallas import tpu as pltpu
from jax.experimental.pallas import tpu_sc as plsc

DIM = 64

NC = 2
NS = 16
NW = NC * NS

SEGS = ((0, 128), (128, 72))
KROWS = 4
NBUF = 2


def _embed(x, table):
    S0, S1 = x.shape
    rows_per_w = S0 // NW
    n_chunks = rows_per_w // KROWS

    mesh = plsc.VectorSubcoreMesh(
        core_axis_name="c", subcore_axis_name="s", num_cores=NC,
        num_subcores=NS)

    @functools.partial(
        pl.kernel,
        out_type=jax.ShapeDtypeStruct((S0, S1, DIM), jnp.float32),
        mesh=mesh,
        compiler_params=pltpu.CompilerParams(use_tc_tiling_on_sc=False),
        scratch_types=[
            pltpu.VMEM((rows_per_w, S1), jnp.int32),
            pltpu.VMEM((NBUF, KROWS, S1, DIM), jnp.float32),
            pltpu.SemaphoreType.DMA,
            pltpu.SemaphoreType.DMA,
            pltpu.SemaphoreType.DMA,
            pltpu.SemaphoreType.DMA,
        ],
    )
    def k(x_hbm, table_hbm, out_hbm, idx_v, rows_v, g0, g1, o0, o1):
        gsems = (g0, g1)
        osems = (o0, o1)
        wid = lax.axis_index("s") * NC + lax.axis_index("c")
        base = pl.multiple_of(wid * rows_per_w, rows_per_w)

        pltpu.sync_copy(x_hbm.at[pl.ds(base, rows_per_w)], idx_v)

        def fire_gathers(i, b):
            for r in range(KROWS):
                for off, n in SEGS:
                    pltpu.async_copy(
                        table_hbm.at[
                            idx_v.at[i * KROWS + r, pl.ds(off, n)]],
                        rows_v.at[b, r, pl.ds(off, n)],
                        gsems[b])

        def wait_gathers(b):
            pltpu.make_async_copy(
                out_hbm.at[pl.ds(0, KROWS)], rows_v.at[b], gsems[b]).wait()

        def fire_out(i, b):
            pltpu.async_copy(
                rows_v.at[b], out_hbm.at[pl.ds(base + i * KROWS, KROWS)],
                osems[b])

        def wait_out(b):
            pltpu.make_async_copy(
                rows_v.at[b], out_hbm.at[pl.ds(0, KROWS)], osems[b]).wait()

        fire_gathers(0, 0)
        fire_gathers(1, 1)
        wait_gathers(0)
        fire_out(0, 0)

        @pl.loop(NBUF, n_chunks, step=NBUF)
        def _(i0):
            for d in range(NBUF):
                i = i0 + d
                b = d
                ob = 1 - b
                wait_out(b)
                fire_gathers(i, b)
                wait_gathers(ob)
                fire_out(i - 1, ob)

        wait_gathers(1)
        fire_out(n_chunks - 1, 1)
        wait_out(0)
        wait_out(1)

    return k(x, table)


def kernel(x, table):
    return _embed(x.astype(jnp.int32), table)
